# baseline (speedup 1.0000x reference)
import jax
import jax.numpy as jnp
from jax import lax
from jax.experimental import pallas as pl
from jax.experimental.pallas import tpu as pltpu
from jax.experimental.pallas import tpu_sc as plsc

N = 10000
NR = 10112
F = 128
H = 128
H3 = 64
CB = 768
E = 320000

BLK = 128
NB = NR // BLK

TBLK = 1264
TNB = NR // TBLK

NCORE = 2
NSUB = 16
TILES = NCORE * NSUB
CHUNK = 88
CH = 114
PT = CH * CHUNK
E2 = TILES * PT
RPS = NR // NSUB
RING = 4
IRING = 5


def _vmesh():
    return plsc.VectorSubcoreMesh(core_axis_name="c", subcore_axis_name="s")


def _sc_deg(idx2):
    def body(idx_hbm, out_hbm, idx_v, ones_v, sv, acc):
        c = lax.axis_index("c")
        s = lax.axis_index("s")
        w = c * NSUB + s
        pltpu.sync_copy(idx_hbm.at[w], idx_v)
        for j in range(128 // 16):
            ones_v[pl.ds(j * 16, 16)] = jnp.ones((16,), jnp.float32)
        for j in range(RPS // 16 + 1):
            sv[pl.ds(j * 16, 16)] = jnp.zeros((16,), jnp.float32)
        r0 = s * RPS
        pltpu.sync_copy(sv.at[pl.ds(0, RPS)], acc.at[pl.ds(r0, RPS)])
        plsc.subcore_barrier()

        def step(k, carry):
            pltpu.sync_copy(ones_v.at[pl.ds(0, CHUNK)],
                            acc.at[idx_v.at[k, 1]], add=True)
            return carry

        lax.fori_loop(0, CH, step, 0)
        plsc.subcore_barrier()
        pltpu.sync_copy(acc.at[pl.ds(r0, RPS)], sv.at[pl.ds(0, RPS)])
        pltpu.sync_copy(sv.at[pl.ds(0, RPS)],
                        out_hbm.at[pl.ds(c * NR + r0, RPS)])

    run = pl.kernel(
        body,
        out_type=jax.ShapeDtypeStruct((NCORE * NR,), jnp.float32),
        mesh=_vmesh(),
        scratch_types=[
            pltpu.VMEM((CH, 2, CHUNK), jnp.int32),
            pltpu.VMEM((128,), jnp.float32),
            pltpu.VMEM((RPS + 16,), jnp.float32),
            pltpu.VMEM_SHARED((NR,), jnp.float32),
        ],
    )
    return run(idx2)


def _sc_scatter(idx2, y, h):
    nstage = (NB + NSUB - 1) // NSUB

    def body(idx_hbm, y_hbm, out_hbm, idxv, buf, acc, sem_i, sem_g, sem_s):
        c = lax.axis_index("c")
        s = lax.axis_index("s")
        w = c * NSUB + s

        def zrow(r, carry):
            for j in range(h // 16):
                buf[r, pl.ds(j * 16, 16)] = jnp.zeros((16,), jnp.float32)
            return carry

        lax.fori_loop(0, BLK, zrow, 0)
        for j in range(nstage):
            blk = s + j * NSUB

            @pl.when(blk < NB)
            def _():
                pltpu.sync_copy(buf.at[pl.ds(0, BLK)],
                                acc.at[pl.ds(blk * BLK, BLK)])
        plsc.subcore_barrier()

        def fetch_idx(k):
            bi = lax.rem(k, IRING)
            pltpu.async_copy(idx_hbm.at[w, k], idxv.at[bi], sem_i.at[bi])

        def wait_idx(k):
            bi = lax.rem(k, IRING)
            pltpu.make_async_copy(idx_hbm.at[w, k], idxv.at[bi],
                                  sem_i.at[bi]).wait()

        def start_gather(k):
            b = lax.rem(k, RING)
            bi = lax.rem(k, IRING)
            pltpu.async_copy(y_hbm.at[idxv.at[bi, 0]],
                             buf.at[pl.ds(b * CHUNK, CHUNK)], sem_g.at[b])

        def wait_gather(k):
            b = lax.rem(k, RING)
            bi = lax.rem(k, IRING)
            pltpu.make_async_copy(y_hbm.at[idxv.at[bi, 0]],
                                  buf.at[pl.ds(b * CHUNK, CHUNK)],
                                  sem_g.at[b]).wait()

        def start_scatter(k):
            b = lax.rem(k, RING)
            bi = lax.rem(k, IRING)
            pltpu.async_copy(buf.at[pl.ds(b * CHUNK, CHUNK)],
                             acc.at[idxv.at[bi, 1]], sem_s.at[b], add=True)

        def wait_scatter(k):
            b = lax.rem(k, RING)
            bi = lax.rem(k, IRING)
            pltpu.make_async_copy(buf.at[pl.ds(b * CHUNK, CHUNK)],
                                  acc.at[idxv.at[bi, 1]], sem_s.at[b]).wait()

        for j in range(3):
            fetch_idx(j)
        for j in range(2):
            wait_idx(j)
            start_gather(j)
        for k in range(2):
            fetch_idx(k + 3)
            wait_idx(k + 2)
            start_gather(k + 2)
            wait_gather(k)
            start_scatter(k)

        def step(k, carry):
            wait_scatter(k - 2)
            fetch_idx(k + 3)
            wait_idx(k + 2)
            start_gather(k + 2)
            wait_gather(k)
            start_scatter(k)
            return carry

        lax.fori_loop(2, CH - 3, step, 0)
        for k in range(CH - 3, CH):
            wait_scatter(k - 2)
            if k + 2 < CH:
                wait_idx(k + 2)
                start_gather(k + 2)
            wait_gather(k)
            start_scatter(k)
        wait_scatter(CH - 2)
        wait_scatter(CH - 1)
        plsc.subcore_barrier()
        for j in range(nstage):
            blk = s + j * NSUB

            @pl.when(blk < NB)
            def _():
                pltpu.sync_copy(acc.at[pl.ds(blk * BLK, BLK)],
                                buf.at[pl.ds(0, BLK)])
                pltpu.sync_copy(buf.at[pl.ds(0, BLK)],
                                out_hbm.at[c, pl.ds(blk * BLK, BLK)])

    run = pl.kernel(
        body,
        out_type=jax.ShapeDtypeStruct((NCORE, NR, h), jnp.float32),
        mesh=_vmesh(),
        compiler_params=(pltpu.CompilerParams(use_tc_tiling_on_sc=False)
                         if h % 128 != 0 else None),
        scratch_types=[
            pltpu.VMEM((IRING, 2, CHUNK), jnp.int32),
            pltpu.VMEM((RING * CHUNK, h), jnp.float32),
            pltpu.VMEM_SHARED((NR, h), jnp.float32),
            pltpu.SemaphoreType.DMA((IRING,)),
            pltpu.SemaphoreType.DMA((RING,)),
            pltpu.SemaphoreType.DMA((RING,)),
        ],
    )
    return run(idx2, y)


def _tc_pre1(xp, d0, d1, W1):
    def body(x_b, d0_b, d1_b, w_b, y_b, dv_b):
        i = pl.program_id(0)
        rows = i * TBLK + lax.broadcasted_iota(jnp.int32, (TBLK, 1), 0)
        d = 1.0 + d0_b[...] + d1_b[...]
        dv = jnp.where(rows < N, lax.rsqrt(d), 0.0)
        dv128 = jnp.broadcast_to(dv, (TBLK, 128))
        dv_b[...] = dv128
        y_b[...] = dv128 * jnp.dot(x_b[...], w_b[...],
                                   preferred_element_type=jnp.float32)

    return pl.pallas_call(
        body,
        grid=(TNB,),
        in_specs=[
            pl.BlockSpec((TBLK, F), lambda i: (i, 0)),
            pl.BlockSpec((TBLK, 1), lambda i: (i, 0)),
            pl.BlockSpec((TBLK, 1), lambda i: (i, 0)),
            pl.BlockSpec((F, H), lambda i: (0, 0)),
        ],
        out_specs=[
            pl.BlockSpec((TBLK, H), lambda i: (i, 0)),
            pl.BlockSpec((TBLK, 128), lambda i: (i, 0)),
        ],
        out_shape=[
            jax.ShapeDtypeStruct((NR, H), jnp.float32),
            jax.ShapeDtypeStruct((NR, 128), jnp.float32),
        ],
    )(xp, d0, d1, W1)


def _tc_mid(pp, y, dinv2, b, g, be, W, hin, hout):
    def body(p_b, y_b, dv_b, b_b, g_b, be_b, w_b, o_b, z_s, st_s):
        ph = pl.program_id(0)
        i = pl.program_id(1)

        @pl.when(ph == 0)
        def _():
            z = dv_b[...][:, :hin] * (p_b[0] + p_b[1] + y_b[...]) + b_b[...]
            z_s[pl.ds(i * TBLK, TBLK), :] = z
            rows = i * TBLK + lax.broadcasted_iota(jnp.int32, (TBLK, 1), 0)
            zm = jnp.where(rows < N, z, 0.0)
            s1 = jnp.sum(zm, axis=0, keepdims=True)
            s2 = jnp.sum(zm * zm, axis=0, keepdims=True)

            @pl.when(i == 0)
            def _():
                st_s[0:1, :hin] = s1
                st_s[1:2, :hin] = s2

            @pl.when(i != 0)
            def _():
                st_s[0:1, :hin] = st_s[0:1, :hin] + s1
                st_s[1:2, :hin] = st_s[1:2, :hin] + s2

        @pl.when(ph == 1)
        def _():
            mu = st_s[0:1, :hin] * (1.0 / N)
            var = st_s[1:2, :hin] * (1.0 / N) - mu * mu
            kk = g_b[...] * lax.rsqrt(var + 1e-5)
            zb = z_s[pl.ds(i * TBLK, TBLK), :]
            hb = jnp.maximum((zb - mu) * kk + be_b[...], 0.0)
            o_b[...] = dv_b[...][:, :hout] * jnp.dot(
                hb, w_b[...], preferred_element_type=jnp.float32)

    return pl.pallas_call(
        body,
        grid=(2, TNB),
        in_specs=[
            pl.BlockSpec((2, TBLK, hin), lambda p, i: (0, i * (1 - p), 0)),
            pl.BlockSpec((TBLK, hin), lambda p, i: (i * (1 - p), 0)),
            pl.BlockSpec((TBLK, 128), lambda p, i: (i, 0)),
            pl.BlockSpec((1, hin), lambda p, i: (0, 0)),
            pl.BlockSpec((1, hin), lambda p, i: (0, 0)),
            pl.BlockSpec((1, hin), lambda p, i: (0, 0)),
            pl.BlockSpec((hin, hout), lambda p, i: (0, 0)),
        ],
        out_specs=pl.BlockSpec((TBLK, hout), lambda p, i: (i * p, 0)),
        out_shape=jax.ShapeDtypeStruct((NR, hout), jnp.float32),
        scratch_shapes=[
            pltpu.VMEM((NR, hin), jnp.float32),
            pltpu.VMEM((8, 128), jnp.float32),
        ],
    )(pp, y, dinv2, b, g, be, W)


def _tc_tail(pp, y3, dinv2, b3, g3, be3, code, Wf1, bf1, Wf2, bf2,
             Wv, bv, Wt, bt, Wc, bc):
    def body(p_b, y_b, dv_b, b_b, g_b, be_b, code_b, wf1_b, bf1_b,
             wf2_b, bf2_b, wv_b, bv_b, wt_b, bt_b, wc_b, bc_b,
             ov, ot, oc, z_s, st_s):
        i = pl.program_id(0)

        @pl.when(i < TNB)
        def _():
            z = dv_b[...][:, :H3] * (p_b[0] + p_b[1] + y_b[...]) + b_b[...]
            z_s[pl.ds(i * TBLK, TBLK), :] = z
            rows = i * TBLK + lax.broadcasted_iota(jnp.int32, (TBLK, 1), 0)
            zm = jnp.where(rows < N, z, 0.0)
            s1 = jnp.sum(zm, axis=0, keepdims=True)
            s2 = jnp.sum(zm * zm, axis=0, keepdims=True)

            @pl.when(i == 0)
            def _():
                st_s[0:1, :H3] = s1
                st_s[1:2, :H3] = s2

            @pl.when(i != 0)
            def _():
                st_s[0:1, :H3] = st_s[0:1, :H3] + s1
                st_s[1:2, :H3] = st_s[1:2, :H3] + s2

        @pl.when(i == TNB)
        def _():
            mu = st_s[0:1, :H3] * (1.0 / N)
            var = st_s[1:2, :H3] * (1.0 / N) - mu * mu
            kk = g_b[...] * lax.rsqrt(var + 1e-5)
            z = z_s[...]
            rows = lax.broadcasted_iota(jnp.int32, (NR, 1), 0)
            hb = jnp.where(rows < N,
                           jnp.maximum((z - mu) * kk + be_b[...], 0.0), 0.0)
            m = jnp.sum(hb, axis=0, keepdims=True) * (1.0 / N)
            f1 = jnp.maximum(
                jnp.dot(m, wf1_b[0:H3, :], preferred_element_type=jnp.float32)
                + jnp.dot(code_b[...], wf1_b[H3:, :],
                          preferred_element_type=jnp.float32)
                + bf1_b[...], 0.0)
            f2 = jnp.maximum(
                jnp.dot(f1, wf2_b[...], preferred_element_type=jnp.float32)
                + bf2_b[...], 0.0)
            ov[...] = jnp.dot(f2, wv_b[...],
                              preferred_element_type=jnp.float32) + bv_b[...]
            ot[...] = jnp.dot(f2, wt_b[...],
                              preferred_element_type=jnp.float32) + bt_b[...]
            lc = jnp.dot(f2, wc_b[...],
                         preferred_element_type=jnp.float32) + bc_b[...]
            oc[...] = 1.0 / (1.0 + jnp.exp(-lc))

    last = TNB - 1
    return pl.pallas_call(
        body,
        grid=(TNB + 1,),
        in_specs=[
            pl.BlockSpec((2, TBLK, H3), lambda i: (0, jnp.minimum(i, last), 0)),
            pl.BlockSpec((TBLK, H3), lambda i: (jnp.minimum(i, last), 0)),
            pl.BlockSpec((TBLK, 128), lambda i: (jnp.minimum(i, last), 0)),
            pl.BlockSpec((1, H3), lambda i: (0, 0)),
            pl.BlockSpec((1, H3), lambda i: (0, 0)),
            pl.BlockSpec((1, H3), lambda i: (0, 0)),
            pl.BlockSpec((1, CB), lambda i: (0, 0)),
            pl.BlockSpec((H3 + CB, 512), lambda i: (0, 0)),
            pl.BlockSpec((1, 512), lambda i: (0, 0)),
            pl.BlockSpec((512, 256), lambda i: (0, 0)),
            pl.BlockSpec((1, 256), lambda i: (0, 0)),
            pl.BlockSpec((256, 2), lambda i: (0, 0)),
            pl.BlockSpec((1, 2), lambda i: (0, 0)),
            pl.BlockSpec((256, 5), lambda i: (0, 0)),
            pl.BlockSpec((1, 5), lambda i: (0, 0)),
            pl.BlockSpec((256, 1), lambda i: (0, 0)),
            pl.BlockSpec((1, 1), lambda i: (0, 0)),
        ],
        out_specs=[
            pl.BlockSpec((1, 2), lambda i: (0, 0)),
            pl.BlockSpec((1, 5), lambda i: (0, 0)),
            pl.BlockSpec((1, 1), lambda i: (0, 0)),
        ],
        out_shape=[
            jax.ShapeDtypeStruct((1, 2), jnp.float32),
            jax.ShapeDtypeStruct((1, 5), jnp.float32),
            jax.ShapeDtypeStruct((1, 1), jnp.float32),
        ],
        scratch_shapes=[
            pltpu.VMEM((NR, H3), jnp.float32),
            pltpu.VMEM((8, 128), jnp.float32),
        ],
    )(pp, y3, dinv2, b3, g3, be3, code, Wf1, bf1, Wf2, bf2,
      Wv, bv, Wt, bt, Wc, bc)


def kernel(x, edge_index, code_embedding,
           W1, b1, g1, be1, W2, b2, g2, be2, W3, b3, g3, be3,
           Wf1, bf1, Wf2, bf2, Wv, bv, Wt, bt, Wc, bc):
    padn = E2 - E
    ar = jnp.arange(padn, dtype=jnp.int32)
    pad_src = (ar * 13) % N
    pad_dst = N + ar % (NR - N)
    src_r = jnp.concatenate([edge_index[0], pad_src]).reshape(TILES, CH, CHUNK)
    dst_r = jnp.concatenate([edge_index[1], pad_dst]).reshape(TILES, CH, CHUNK)
    idx2 = jnp.stack([src_r, dst_r], axis=2)

    xp = jnp.pad(x, ((0, NR - N), (0, 0)))

    degp = _sc_deg(idx2).reshape(NCORE, NR)
    d0 = degp[0].reshape(NR, 1)
    d1 = degp[1].reshape(NR, 1)

    y1, dinv2 = _tc_pre1(xp, d0, d1, W1)

    p1 = _sc_scatter(idx2, y1, H)
    y2 = _tc_mid(p1, y1, dinv2, b1.reshape(1, H), g1.reshape(1, H),
                 be1.reshape(1, H), W2, H, H)

    p2 = _sc_scatter(idx2, y2, H)
    y3 = _tc_mid(p2, y2, dinv2, b2.reshape(1, H), g2.reshape(1, H),
                 be2.reshape(1, H), W3, H, H3)

    p3 = _sc_scatter(idx2, y3, H3)
    vuln, typ, conf = _tc_tail(
        p3, y3, dinv2, b3.reshape(1, H3), g3.reshape(1, H3),
        be3.reshape(1, H3), code_embedding, Wf1, bf1.reshape(1, 512),
        Wf2, bf2.reshape(1, 256), Wv, bv.reshape(1, 2),
        Wt, bt.reshape(1, 5), Wc, bc.reshape(1, 1))
    return (vuln, typ, conf)


# speedup vs baseline: 1.0378x; 1.0378x over previous
import jax
import jax.numpy as jnp
from jax import lax
from jax.experimental import pallas as pl
from jax.experimental.pallas import tpu as pltpu
from jax.experimental.pallas import tpu_sc as plsc

N = 10000
NR = 10112
F = 128
H = 128
H3 = 64
CB = 768
E = 320000

BLK = 128
NB = NR // BLK

TBLK = 1264
TNB = NR // TBLK

NCORE = 2
NSUB = 16
TILES = NCORE * NSUB
CHUNK = 120
CH = 84
PT = CH * CHUNK
E2 = TILES * PT
RPS = NR // NSUB
RING = 3
IRING = 6


def _vmesh():
    return plsc.VectorSubcoreMesh(core_axis_name="c", subcore_axis_name="s")


def _sc_deg(idx2):
    def body(idx_hbm, out_hbm, idx_v, ones_v, sv, acc):
        c = lax.axis_index("c")
        s = lax.axis_index("s")
        w = c * NSUB + s
        pltpu.sync_copy(idx_hbm.at[w], idx_v)
        for j in range(128 // 16):
            ones_v[pl.ds(j * 16, 16)] = jnp.ones((16,), jnp.float32)
        for j in range(RPS // 16 + 1):
            sv[pl.ds(j * 16, 16)] = jnp.zeros((16,), jnp.float32)
        r0 = s * RPS
        pltpu.sync_copy(sv.at[pl.ds(0, RPS)], acc.at[pl.ds(r0, RPS)])
        plsc.subcore_barrier()

        def step(k, carry):
            pltpu.sync_copy(ones_v.at[pl.ds(0, CHUNK)],
                            acc.at[idx_v.at[k, 1]], add=True)
            return carry

        lax.fori_loop(0, CH, step, 0)
        plsc.subcore_barrier()
        pltpu.sync_copy(acc.at[pl.ds(r0, RPS)], sv.at[pl.ds(0, RPS)])
        pltpu.sync_copy(sv.at[pl.ds(0, RPS)],
                        out_hbm.at[pl.ds(c * NR + r0, RPS)])

    run = pl.kernel(
        body,
        out_type=jax.ShapeDtypeStruct((NCORE * NR,), jnp.float32),
        mesh=_vmesh(),
        scratch_types=[
            pltpu.VMEM((CH, 2, CHUNK), jnp.int32),
            pltpu.VMEM((128,), jnp.float32),
            pltpu.VMEM((RPS + 16,), jnp.float32),
            pltpu.VMEM_SHARED((NR,), jnp.float32),
        ],
    )
    return run(idx2)


def _sc_scatter(idx2, y, h):
    nstage = (NB + NSUB - 1) // NSUB

    def body(idx_hbm, y_hbm, out_hbm, idxv, buf, acc, sem_i, sem_g, sem_s):
        c = lax.axis_index("c")
        s = lax.axis_index("s")
        w = c * NSUB + s

        def zrow(r, carry):
            for j in range(h // 16):
                buf[r, pl.ds(j * 16, 16)] = jnp.zeros((16,), jnp.float32)
            return carry

        lax.fori_loop(0, BLK, zrow, 0)
        for j in range(nstage):
            blk = s + j * NSUB

            @pl.when(blk < NB)
            def _():
                pltpu.sync_copy(buf.at[pl.ds(0, BLK)],
                                acc.at[pl.ds(blk * BLK, BLK)])
        plsc.subcore_barrier()

        def fetch_idx(k):
            bi = lax.rem(k, IRING)
            pltpu.async_copy(idx_hbm.at[w, k], idxv.at[bi], sem_i.at[bi])

        def wait_idx(k):
            bi = lax.rem(k, IRING)
            pltpu.make_async_copy(idx_hbm.at[w, k], idxv.at[bi],
                                  sem_i.at[bi]).wait()

        def start_gather(k):
            b = lax.rem(k, RING)
            bi = lax.rem(k, IRING)
            pltpu.async_copy(y_hbm.at[idxv.at[bi, 0]],
                             buf.at[pl.ds(b * CHUNK, CHUNK)], sem_g.at[b])

        def wait_gather(k):
            b = lax.rem(k, RING)
            bi = lax.rem(k, IRING)
            pltpu.make_async_copy(y_hbm.at[idxv.at[bi, 0]],
                                  buf.at[pl.ds(b * CHUNK, CHUNK)],
                                  sem_g.at[b]).wait()

        def start_scatter(k):
            b = lax.rem(k, RING)
            bi = lax.rem(k, IRING)
            pltpu.async_copy(buf.at[pl.ds(b * CHUNK, CHUNK)],
                             acc.at[idxv.at[bi, 1]], sem_s.at[b], add=True)

        def wait_scatter(k):
            b = lax.rem(k, RING)
            bi = lax.rem(k, IRING)
            pltpu.make_async_copy(buf.at[pl.ds(b * CHUNK, CHUNK)],
                                  acc.at[idxv.at[bi, 1]], sem_s.at[b]).wait()

        for j in range(4):
            fetch_idx(j)
        for j in range(2):
            wait_idx(j)
            start_gather(j)
        fetch_idx(4)
        wait_gather(0)
        start_scatter(0)
        wait_idx(2)
        start_gather(2)

        def step(k, carry):
            fetch_idx(k + 4)
            wait_gather(k)
            wait_scatter(k - 1)
            wait_idx(k + 2)
            start_gather(k + 2)
            start_scatter(k)
            return carry

        lax.fori_loop(1, CH - 4, step, 0)
        for k in range(CH - 4, CH):
            wait_gather(k)
            start_scatter(k)
            wait_scatter(k - 1)
            if k + 2 < CH:
                wait_idx(k + 2)
                start_gather(k + 2)
        wait_scatter(CH - 1)
        plsc.subcore_barrier()
        for j in range(nstage):
            blk = s + j * NSUB

            @pl.when(blk < NB)
            def _():
                pltpu.sync_copy(acc.at[pl.ds(blk * BLK, BLK)],
                                buf.at[pl.ds(0, BLK)])
                pltpu.sync_copy(buf.at[pl.ds(0, BLK)],
                                out_hbm.at[c, pl.ds(blk * BLK, BLK)])

    run = pl.kernel(
        body,
        out_type=jax.ShapeDtypeStruct((NCORE, NR, h), jnp.float32),
        mesh=_vmesh(),
        compiler_params=(pltpu.CompilerParams(use_tc_tiling_on_sc=False)
                         if h % 128 != 0 else None),
        scratch_types=[
            pltpu.VMEM((IRING, 2, CHUNK), jnp.int32),
            pltpu.VMEM((RING * CHUNK, h), jnp.float32),
            pltpu.VMEM_SHARED((NR, h), jnp.float32),
            pltpu.SemaphoreType.DMA((IRING,)),
            pltpu.SemaphoreType.DMA((RING,)),
            pltpu.SemaphoreType.DMA((RING,)),
        ],
    )
    return run(idx2, y)


def _tc_pre1(xp, d0, d1, W1):
    def body(x_b, d0_b, d1_b, w_b, y_b, dv_b):
        i = pl.program_id(0)
        rows = i * TBLK + lax.broadcasted_iota(jnp.int32, (TBLK, 1), 0)
        d = 1.0 + d0_b[...] + d1_b[...]
        dv = jnp.where(rows < N, lax.rsqrt(d), 0.0)
        dv128 = jnp.broadcast_to(dv, (TBLK, 128))
        dv_b[...] = dv128
        y_b[...] = dv128 * jnp.dot(x_b[...], w_b[...],
                                   preferred_element_type=jnp.float32)

    return pl.pallas_call(
        body,
        grid=(TNB,),
        in_specs=[
            pl.BlockSpec((TBLK, F), lambda i: (i, 0)),
            pl.BlockSpec((TBLK, 1), lambda i: (i, 0)),
            pl.BlockSpec((TBLK, 1), lambda i: (i, 0)),
            pl.BlockSpec((F, H), lambda i: (0, 0)),
        ],
        out_specs=[
            pl.BlockSpec((TBLK, H), lambda i: (i, 0)),
            pl.BlockSpec((TBLK, 128), lambda i: (i, 0)),
        ],
        out_shape=[
            jax.ShapeDtypeStruct((NR, H), jnp.float32),
            jax.ShapeDtypeStruct((NR, 128), jnp.float32),
        ],
    )(xp, d0, d1, W1)


def _tc_mid(pp, y, dinv2, b, g, be, W, hin, hout):
    def body(p_b, y_b, dv_b, b_b, g_b, be_b, w_b, o_b, z_s, st_s):
        ph = pl.program_id(0)
        i = pl.program_id(1)

        @pl.when(ph == 0)
        def _():
            z = dv_b[...][:, :hin] * (p_b[0] + p_b[1] + y_b[...]) + b_b[...]
            z_s[pl.ds(i * TBLK, TBLK), :] = z
            rows = i * TBLK + lax.broadcasted_iota(jnp.int32, (TBLK, 1), 0)
            zm = jnp.where(rows < N, z, 0.0)
            s1 = jnp.sum(zm, axis=0, keepdims=True)
            s2 = jnp.sum(zm * zm, axis=0, keepdims=True)

            @pl.when(i == 0)
            def _():
                st_s[0:1, :hin] = s1
                st_s[1:2, :hin] = s2

            @pl.when(i != 0)
            def _():
                st_s[0:1, :hin] = st_s[0:1, :hin] + s1
                st_s[1:2, :hin] = st_s[1:2, :hin] + s2

        @pl.when(ph == 1)
        def _():
            mu = st_s[0:1, :hin] * (1.0 / N)
            var = st_s[1:2, :hin] * (1.0 / N) - mu * mu
            kk = g_b[...] * lax.rsqrt(var + 1e-5)
            zb = z_s[pl.ds(i * TBLK, TBLK), :]
            hb = jnp.maximum((zb - mu) * kk + be_b[...], 0.0)
            o_b[...] = dv_b[...][:, :hout] * jnp.dot(
                hb, w_b[...], preferred_element_type=jnp.float32)

    return pl.pallas_call(
        body,
        grid=(2, TNB),
        in_specs=[
            pl.BlockSpec((2, TBLK, hin), lambda p, i: (0, i * (1 - p), 0)),
            pl.BlockSpec((TBLK, hin), lambda p, i: (i * (1 - p), 0)),
            pl.BlockSpec((TBLK, 128), lambda p, i: (i, 0)),
            pl.BlockSpec((1, hin), lambda p, i: (0, 0)),
            pl.BlockSpec((1, hin), lambda p, i: (0, 0)),
            pl.BlockSpec((1, hin), lambda p, i: (0, 0)),
            pl.BlockSpec((hin, hout), lambda p, i: (0, 0)),
        ],
        out_specs=pl.BlockSpec((TBLK, hout), lambda p, i: (i * p, 0)),
        out_shape=jax.ShapeDtypeStruct((NR, hout), jnp.float32),
        scratch_shapes=[
            pltpu.VMEM((NR, hin), jnp.float32),
            pltpu.VMEM((8, 128), jnp.float32),
        ],
    )(pp, y, dinv2, b, g, be, W)


def _tc_tail(pp, y3, dinv2, b3, g3, be3, code, Wf1, bf1, Wf2, bf2,
             Wv, bv, Wt, bt, Wc, bc):
    def body(p_b, y_b, dv_b, b_b, g_b, be_b, code_b, wf1_b, bf1_b,
             wf2_b, bf2_b, wv_b, bv_b, wt_b, bt_b, wc_b, bc_b,
             ov, ot, oc, z_s, st_s):
        i = pl.program_id(0)

        @pl.when(i < TNB)
        def _():
            z = dv_b[...][:, :H3] * (p_b[0] + p_b[1] + y_b[...]) + b_b[...]
            z_s[pl.ds(i * TBLK, TBLK), :] = z
            rows = i * TBLK + lax.broadcasted_iota(jnp.int32, (TBLK, 1), 0)
            zm = jnp.where(rows < N, z, 0.0)
            s1 = jnp.sum(zm, axis=0, keepdims=True)
            s2 = jnp.sum(zm * zm, axis=0, keepdims=True)

            @pl.when(i == 0)
            def _():
                st_s[0:1, :H3] = s1
                st_s[1:2, :H3] = s2

            @pl.when(i != 0)
            def _():
                st_s[0:1, :H3] = st_s[0:1, :H3] + s1
                st_s[1:2, :H3] = st_s[1:2, :H3] + s2

        @pl.when(i == TNB)
        def _():
            mu = st_s[0:1, :H3] * (1.0 / N)
            var = st_s[1:2, :H3] * (1.0 / N) - mu * mu
            kk = g_b[...] * lax.rsqrt(var + 1e-5)
            z = z_s[...]
            rows = lax.broadcasted_iota(jnp.int32, (NR, 1), 0)
            hb = jnp.where(rows < N,
                           jnp.maximum((z - mu) * kk + be_b[...], 0.0), 0.0)
            m = jnp.sum(hb, axis=0, keepdims=True) * (1.0 / N)
            f1 = jnp.maximum(
                jnp.dot(m, wf1_b[0:H3, :], preferred_element_type=jnp.float32)
                + jnp.dot(code_b[...], wf1_b[H3:, :],
                          preferred_element_type=jnp.float32)
                + bf1_b[...], 0.0)
            f2 = jnp.maximum(
                jnp.dot(f1, wf2_b[...], preferred_element_type=jnp.float32)
                + bf2_b[...], 0.0)
            ov[...] = jnp.dot(f2, wv_b[...],
                              preferred_element_type=jnp.float32) + bv_b[...]
            ot[...] = jnp.dot(f2, wt_b[...],
                              preferred_element_type=jnp.float32) + bt_b[...]
            lc = jnp.dot(f2, wc_b[...],
                         preferred_element_type=jnp.float32) + bc_b[...]
            oc[...] = 1.0 / (1.0 + jnp.exp(-lc))

    last = TNB - 1
    return pl.pallas_call(
        body,
        grid=(TNB + 1,),
        in_specs=[
            pl.BlockSpec((2, TBLK, H3), lambda i: (0, jnp.minimum(i, last), 0)),
            pl.BlockSpec((TBLK, H3), lambda i: (jnp.minimum(i, last), 0)),
            pl.BlockSpec((TBLK, 128), lambda i: (jnp.minimum(i, last), 0)),
            pl.BlockSpec((1, H3), lambda i: (0, 0)),
            pl.BlockSpec((1, H3), lambda i: (0, 0)),
            pl.BlockSpec((1, H3), lambda i: (0, 0)),
            pl.BlockSpec((1, CB), lambda i: (0, 0)),
            pl.BlockSpec((H3 + CB, 512), lambda i: (0, 0)),
            pl.BlockSpec((1, 512), lambda i: (0, 0)),
            pl.BlockSpec((512, 256), lambda i: (0, 0)),
            pl.BlockSpec((1, 256), lambda i: (0, 0)),
            pl.BlockSpec((256, 2), lambda i: (0, 0)),
            pl.BlockSpec((1, 2), lambda i: (0, 0)),
            pl.BlockSpec((256, 5), lambda i: (0, 0)),
            pl.BlockSpec((1, 5), lambda i: (0, 0)),
            pl.BlockSpec((256, 1), lambda i: (0, 0)),
            pl.BlockSpec((1, 1), lambda i: (0, 0)),
        ],
        out_specs=[
            pl.BlockSpec((1, 2), lambda i: (0, 0)),
            pl.BlockSpec((1, 5), lambda i: (0, 0)),
            pl.BlockSpec((1, 1), lambda i: (0, 0)),
        ],
        out_shape=[
            jax.ShapeDtypeStruct((1, 2), jnp.float32),
            jax.ShapeDtypeStruct((1, 5), jnp.float32),
            jax.ShapeDtypeStruct((1, 1), jnp.float32),
        ],
        scratch_shapes=[
            pltpu.VMEM((NR, H3), jnp.float32),
            pltpu.VMEM((8, 128), jnp.float32),
        ],
    )(pp, y3, dinv2, b3, g3, be3, code, Wf1, bf1, Wf2, bf2,
      Wv, bv, Wt, bt, Wc, bc)


def kernel(x, edge_index, code_embedding,
           W1, b1, g1, be1, W2, b2, g2, be2, W3, b3, g3, be3,
           Wf1, bf1, Wf2, bf2, Wv, bv, Wt, bt, Wc, bc):
    padn = E2 - E
    ar = jnp.arange(padn, dtype=jnp.int32)
    pad_src = (ar * 13) % N
    pad_dst = N + ar % (NR - N)
    src_r = jnp.concatenate([edge_index[0], pad_src]).reshape(TILES, CH, CHUNK)
    dst_r = jnp.concatenate([edge_index[1], pad_dst]).reshape(TILES, CH, CHUNK)
    idx2 = jnp.stack([src_r, dst_r], axis=2)

    xp = jnp.pad(x, ((0, NR - N), (0, 0)))

    degp = _sc_deg(idx2).reshape(NCORE, NR)
    d0 = degp[0].reshape(NR, 1)
    d1 = degp[1].reshape(NR, 1)

    y1, dinv2 = _tc_pre1(xp, d0, d1, W1)

    p1 = _sc_scatter(idx2, y1, H)
    y2 = _tc_mid(p1, y1, dinv2, b1.reshape(1, H), g1.reshape(1, H),
                 be1.reshape(1, H), W2, H, H)

    p2 = _sc_scatter(idx2, y2, H)
    y3 = _tc_mid(p2, y2, dinv2, b2.reshape(1, H), g2.reshape(1, H),
                 be2.reshape(1, H), W3, H, H3)

    p3 = _sc_scatter(idx2, y3, H3)
    vuln, typ, conf = _tc_tail(
        p3, y3, dinv2, b3.reshape(1, H3), g3.reshape(1, H3),
        be3.reshape(1, H3), code_embedding, Wf1, bf1.reshape(1, 512),
        Wf2, bf2.reshape(1, 256), Wv, bv.reshape(1, 2),
        Wt, bt.reshape(1, 5), Wc, bc.reshape(1, 1))
    return (vuln, typ, conf)


# speedup vs baseline: 1.0866x; 1.0470x over previous
import jax
import jax.numpy as jnp
from jax import lax
from jax.experimental import pallas as pl
from jax.experimental.pallas import tpu as pltpu
from jax.experimental.pallas import tpu_sc as plsc

N = 10000
NR = 10112
F = 128
H = 128
H3 = 64
CB = 768
E = 320000

BLK = 128
NB = NR // BLK

TBLK = 1264
TNB = NR // TBLK

NCORE = 2
NSUB = 16
TILES = NCORE * NSUB
CHUNK = 120
CH = 84
PT = CH * CHUNK
E2 = TILES * PT
RPS = NR // NSUB
RING = 3
IRING = 6


def _vmesh():
    return plsc.VectorSubcoreMesh(core_axis_name="c", subcore_axis_name="s")


def _sc_deg(idx2):
    def body(idx_hbm, out_hbm, idx_v, ones_v, sv, acc):
        c = lax.axis_index("c")
        s = lax.axis_index("s")
        w = c * NSUB + s
        pltpu.sync_copy(idx_hbm.at[1, w], idx_v)
        for j in range(128 // 16):
            ones_v[pl.ds(j * 16, 16)] = jnp.ones((16,), jnp.float32)
        for j in range(RPS // 16 + 1):
            sv[pl.ds(j * 16, 16)] = jnp.zeros((16,), jnp.float32)
        r0 = s * RPS
        pltpu.sync_copy(sv.at[pl.ds(0, RPS)], acc.at[pl.ds(r0, RPS)])
        plsc.subcore_barrier()

        def step(k, carry):
            pltpu.sync_copy(ones_v.at[pl.ds(0, CHUNK)],
                            acc.at[idx_v.at[k]], add=True)
            return carry

        lax.fori_loop(0, CH, step, 0)
        plsc.subcore_barrier()
        pltpu.sync_copy(acc.at[pl.ds(r0, RPS)], sv.at[pl.ds(0, RPS)])
        pltpu.sync_copy(sv.at[pl.ds(0, RPS)],
                        out_hbm.at[pl.ds(c * NR + r0, RPS)])

    run = pl.kernel(
        body,
        out_type=jax.ShapeDtypeStruct((NCORE * NR,), jnp.float32),
        mesh=_vmesh(),
        scratch_types=[
            pltpu.VMEM((CH, CHUNK), jnp.int32),
            pltpu.VMEM((128,), jnp.float32),
            pltpu.VMEM((RPS + 16,), jnp.float32),
            pltpu.VMEM_SHARED((NR,), jnp.float32),
        ],
    )
    return run(idx2)


def _sc_scatter(idx2, y, h):
    nstage = (NB + NSUB - 1) // NSUB

    def body(idx_hbm, y_hbm, out_hbm, idxv, buf, acc, sem_i, sem_g, sem_s):
        c = lax.axis_index("c")
        s = lax.axis_index("s")
        w = c * NSUB + s

        def zrow(r, carry):
            for j in range(h // 16):
                buf[r, pl.ds(j * 16, 16)] = jnp.zeros((16,), jnp.float32)
            return carry

        lax.fori_loop(0, BLK, zrow, 0)
        for j in range(nstage):
            blk = s + j * NSUB

            @pl.when(blk < NB)
            def _():
                pltpu.sync_copy(buf.at[pl.ds(0, BLK)],
                                acc.at[pl.ds(blk * BLK, BLK)])
        plsc.subcore_barrier()

        def fetch_idx(k):
            bi = lax.rem(k, IRING)
            pltpu.async_copy(idx_hbm.at[:, w, k], idxv.at[bi], sem_i.at[bi])

        def wait_idx(k):
            bi = lax.rem(k, IRING)
            pltpu.make_async_copy(idx_hbm.at[:, w, k], idxv.at[bi],
                                  sem_i.at[bi]).wait()

        def start_gather(k):
            b = lax.rem(k, RING)
            bi = lax.rem(k, IRING)
            pltpu.async_copy(y_hbm.at[idxv.at[bi, 0]],
                             buf.at[pl.ds(b * CHUNK, CHUNK)], sem_g.at[b])

        def wait_gather(k):
            b = lax.rem(k, RING)
            bi = lax.rem(k, IRING)
            pltpu.make_async_copy(y_hbm.at[idxv.at[bi, 0]],
                                  buf.at[pl.ds(b * CHUNK, CHUNK)],
                                  sem_g.at[b]).wait()

        def start_scatter(k):
            b = lax.rem(k, RING)
            bi = lax.rem(k, IRING)
            pltpu.async_copy(buf.at[pl.ds(b * CHUNK, CHUNK)],
                             acc.at[idxv.at[bi, 1]], sem_s.at[b], add=True)

        def wait_scatter(k):
            b = lax.rem(k, RING)
            bi = lax.rem(k, IRING)
            pltpu.make_async_copy(buf.at[pl.ds(b * CHUNK, CHUNK)],
                                  acc.at[idxv.at[bi, 1]], sem_s.at[b]).wait()

        for j in range(4):
            fetch_idx(j)
        for j in range(2):
            wait_idx(j)
            start_gather(j)
        fetch_idx(4)
        wait_gather(0)
        start_scatter(0)
        wait_idx(2)
        start_gather(2)

        def step(k, carry):
            fetch_idx(k + 4)
            wait_gather(k)
            wait_scatter(k - 1)
            wait_idx(k + 2)
            start_gather(k + 2)
            start_scatter(k)
            return carry

        lax.fori_loop(1, CH - 4, step, 0)
        for k in range(CH - 4, CH):
            wait_gather(k)
            start_scatter(k)
            wait_scatter(k - 1)
            if k + 2 < CH:
                wait_idx(k + 2)
                start_gather(k + 2)
        wait_scatter(CH - 1)
        plsc.subcore_barrier()
        for j in range(nstage):
            blk = s + j * NSUB

            @pl.when(blk < NB)
            def _():
                pltpu.sync_copy(acc.at[pl.ds(blk * BLK, BLK)],
                                buf.at[pl.ds(0, BLK)])
                pltpu.sync_copy(buf.at[pl.ds(0, BLK)],
                                out_hbm.at[c, pl.ds(blk * BLK, BLK)])

    run = pl.kernel(
        body,
        out_type=jax.ShapeDtypeStruct((NCORE, NR, h), jnp.float32),
        mesh=_vmesh(),
        compiler_params=(pltpu.CompilerParams(use_tc_tiling_on_sc=False)
                         if h % 128 != 0 else None),
        scratch_types=[
            pltpu.VMEM((IRING, 2, CHUNK), jnp.int32),
            pltpu.VMEM((RING * CHUNK, h), jnp.float32),
            pltpu.VMEM_SHARED((NR, h), jnp.float32),
            pltpu.SemaphoreType.DMA((IRING,)),
            pltpu.SemaphoreType.DMA((RING,)),
            pltpu.SemaphoreType.DMA((RING,)),
        ],
    )
    return run(idx2, y)


def _tc_pre1(xp, d0, d1, W1):
    def body(x_b, d0_b, d1_b, w_b, y_b, dv_b):
        i = pl.program_id(0)
        rows = i * TBLK + lax.broadcasted_iota(jnp.int32, (TBLK, 1), 0)
        d = 1.0 + d0_b[...] + d1_b[...]
        dv = jnp.where(rows < N, lax.rsqrt(d), 0.0)
        dv128 = jnp.broadcast_to(dv, (TBLK, 128))
        dv_b[...] = dv128
        y_b[...] = dv128 * jnp.dot(x_b[...], w_b[...],
                                   preferred_element_type=jnp.float32)

    return pl.pallas_call(
        body,
        grid=(TNB,),
        in_specs=[
            pl.BlockSpec((TBLK, F), lambda i: (i, 0)),
            pl.BlockSpec((TBLK, 1), lambda i: (i, 0)),
            pl.BlockSpec((TBLK, 1), lambda i: (i, 0)),
            pl.BlockSpec((F, H), lambda i: (0, 0)),
        ],
        out_specs=[
            pl.BlockSpec((TBLK, H), lambda i: (i, 0)),
            pl.BlockSpec((TBLK, 128), lambda i: (i, 0)),
        ],
        out_shape=[
            jax.ShapeDtypeStruct((NR, H), jnp.float32),
            jax.ShapeDtypeStruct((NR, 128), jnp.float32),
        ],
    )(xp, d0, d1, W1)


def _tc_mid(pp, y, dinv2, b, g, be, W, hin, hout):
    def body(p_b, y_b, dv_b, b_b, g_b, be_b, w_b, o_b, z_s, st_s):
        ph = pl.program_id(0)
        i = pl.program_id(1)

        @pl.when(ph == 0)
        def _():
            z = dv_b[...][:, :hin] * (p_b[0] + p_b[1] + y_b[...]) + b_b[...]
            z_s[pl.ds(i * TBLK, TBLK), :] = z
            rows = i * TBLK + lax.broadcasted_iota(jnp.int32, (TBLK, 1), 0)
            zm = jnp.where(rows < N, z, 0.0)
            s1 = jnp.sum(zm, axis=0, keepdims=True)
            s2 = jnp.sum(zm * zm, axis=0, keepdims=True)

            @pl.when(i == 0)
            def _():
                st_s[0:1, :hin] = s1
                st_s[1:2, :hin] = s2

            @pl.when(i != 0)
            def _():
                st_s[0:1, :hin] = st_s[0:1, :hin] + s1
                st_s[1:2, :hin] = st_s[1:2, :hin] + s2

        @pl.when(ph == 1)
        def _():
            mu = st_s[0:1, :hin] * (1.0 / N)
            var = st_s[1:2, :hin] * (1.0 / N) - mu * mu
            kk = g_b[...] * lax.rsqrt(var + 1e-5)
            zb = z_s[pl.ds(i * TBLK, TBLK), :]
            hb = jnp.maximum((zb - mu) * kk + be_b[...], 0.0)
            o_b[...] = dv_b[...][:, :hout] * jnp.dot(
                hb, w_b[...], preferred_element_type=jnp.float32)

    return pl.pallas_call(
        body,
        grid=(2, TNB),
        in_specs=[
            pl.BlockSpec((2, TBLK, hin), lambda p, i: (0, i * (1 - p), 0)),
            pl.BlockSpec((TBLK, hin), lambda p, i: (i * (1 - p), 0)),
            pl.BlockSpec((TBLK, 128), lambda p, i: (i, 0)),
            pl.BlockSpec((1, hin), lambda p, i: (0, 0)),
            pl.BlockSpec((1, hin), lambda p, i: (0, 0)),
            pl.BlockSpec((1, hin), lambda p, i: (0, 0)),
            pl.BlockSpec((hin, hout), lambda p, i: (0, 0)),
        ],
        out_specs=pl.BlockSpec((TBLK, hout), lambda p, i: (i * p, 0)),
        out_shape=jax.ShapeDtypeStruct((NR, hout), jnp.float32),
        scratch_shapes=[
            pltpu.VMEM((NR, hin), jnp.float32),
            pltpu.VMEM((8, 128), jnp.float32),
        ],
    )(pp, y, dinv2, b, g, be, W)


def _tc_tail(pp, y3, dinv2, b3, g3, be3, code, Wf1, bf1, Wf2, bf2,
             Wv, bv, Wt, bt, Wc, bc):
    def body(p_b, y_b, dv_b, b_b, g_b, be_b, code_b, wf1_b, bf1_b,
             wf2_b, bf2_b, wv_b, bv_b, wt_b, bt_b, wc_b, bc_b,
             ov, ot, oc, z_s, st_s):
        i = pl.program_id(0)

        @pl.when(i < TNB)
        def _():
            z = dv_b[...][:, :H3] * (p_b[0] + p_b[1] + y_b[...]) + b_b[...]
            z_s[pl.ds(i * TBLK, TBLK), :] = z
            rows = i * TBLK + lax.broadcasted_iota(jnp.int32, (TBLK, 1), 0)
            zm = jnp.where(rows < N, z, 0.0)
            s1 = jnp.sum(zm, axis=0, keepdims=True)
            s2 = jnp.sum(zm * zm, axis=0, keepdims=True)

            @pl.when(i == 0)
            def _():
                st_s[0:1, :H3] = s1
                st_s[1:2, :H3] = s2

            @pl.when(i != 0)
            def _():
                st_s[0:1, :H3] = st_s[0:1, :H3] + s1
                st_s[1:2, :H3] = st_s[1:2, :H3] + s2

        @pl.when(i == TNB)
        def _():
            mu = st_s[0:1, :H3] * (1.0 / N)
            var = st_s[1:2, :H3] * (1.0 / N) - mu * mu
            kk = g_b[...] * lax.rsqrt(var + 1e-5)
            z = z_s[...]
            rows = lax.broadcasted_iota(jnp.int32, (NR, 1), 0)
            hb = jnp.where(rows < N,
                           jnp.maximum((z - mu) * kk + be_b[...], 0.0), 0.0)
            m = jnp.sum(hb, axis=0, keepdims=True) * (1.0 / N)
            f1 = jnp.maximum(
                jnp.dot(m, wf1_b[0:H3, :], preferred_element_type=jnp.float32)
                + jnp.dot(code_b[...], wf1_b[H3:, :],
                          preferred_element_type=jnp.float32)
                + bf1_b[...], 0.0)
            f2 = jnp.maximum(
                jnp.dot(f1, wf2_b[...], preferred_element_type=jnp.float32)
                + bf2_b[...], 0.0)
            ov[...] = jnp.dot(f2, wv_b[...],
                              preferred_element_type=jnp.float32) + bv_b[...]
            ot[...] = jnp.dot(f2, wt_b[...],
                              preferred_element_type=jnp.float32) + bt_b[...]
            lc = jnp.dot(f2, wc_b[...],
                         preferred_element_type=jnp.float32) + bc_b[...]
            oc[...] = 1.0 / (1.0 + jnp.exp(-lc))

    last = TNB - 1
    return pl.pallas_call(
        body,
        grid=(TNB + 1,),
        in_specs=[
            pl.BlockSpec((2, TBLK, H3), lambda i: (0, jnp.minimum(i, last), 0)),
            pl.BlockSpec((TBLK, H3), lambda i: (jnp.minimum(i, last), 0)),
            pl.BlockSpec((TBLK, 128), lambda i: (jnp.minimum(i, last), 0)),
            pl.BlockSpec((1, H3), lambda i: (0, 0)),
            pl.BlockSpec((1, H3), lambda i: (0, 0)),
            pl.BlockSpec((1, H3), lambda i: (0, 0)),
            pl.BlockSpec((1, CB), lambda i: (0, 0)),
            pl.BlockSpec((H3 + CB, 512), lambda i: (0, 0)),
            pl.BlockSpec((1, 512), lambda i: (0, 0)),
            pl.BlockSpec((512, 256), lambda i: (0, 0)),
            pl.BlockSpec((1, 256), lambda i: (0, 0)),
            pl.BlockSpec((256, 2), lambda i: (0, 0)),
            pl.BlockSpec((1, 2), lambda i: (0, 0)),
            pl.BlockSpec((256, 5), lambda i: (0, 0)),
            pl.BlockSpec((1, 5), lambda i: (0, 0)),
            pl.BlockSpec((256, 1), lambda i: (0, 0)),
            pl.BlockSpec((1, 1), lambda i: (0, 0)),
        ],
        out_specs=[
            pl.BlockSpec((1, 2), lambda i: (0, 0)),
            pl.BlockSpec((1, 5), lambda i: (0, 0)),
            pl.BlockSpec((1, 1), lambda i: (0, 0)),
        ],
        out_shape=[
            jax.ShapeDtypeStruct((1, 2), jnp.float32),
            jax.ShapeDtypeStruct((1, 5), jnp.float32),
            jax.ShapeDtypeStruct((1, 1), jnp.float32),
        ],
        scratch_shapes=[
            pltpu.VMEM((NR, H3), jnp.float32),
            pltpu.VMEM((8, 128), jnp.float32),
        ],
    )(pp, y3, dinv2, b3, g3, be3, code, Wf1, bf1, Wf2, bf2,
      Wv, bv, Wt, bt, Wc, bc)


def kernel(x, edge_index, code_embedding,
           W1, b1, g1, be1, W2, b2, g2, be2, W3, b3, g3, be3,
           Wf1, bf1, Wf2, bf2, Wv, bv, Wt, bt, Wc, bc):
    padn = E2 - E
    ar = jnp.arange(padn, dtype=jnp.int32)
    pad_src = (ar * 13) % N
    pad_dst = N + ar % (NR - N)
    pad2 = jnp.stack([pad_src, pad_dst])
    idx2 = jnp.concatenate([edge_index, pad2], axis=1).reshape(
        2, TILES, CH, CHUNK)

    xp = jnp.pad(x, ((0, NR - N), (0, 0)))

    degp = _sc_deg(idx2).reshape(NCORE, NR)
    d0 = degp[0].reshape(NR, 1)
    d1 = degp[1].reshape(NR, 1)

    y1, dinv2 = _tc_pre1(xp, d0, d1, W1)

    p1 = _sc_scatter(idx2, y1, H)
    y2 = _tc_mid(p1, y1, dinv2, b1.reshape(1, H), g1.reshape(1, H),
                 be1.reshape(1, H), W2, H, H)

    p2 = _sc_scatter(idx2, y2, H)
    y3 = _tc_mid(p2, y2, dinv2, b2.reshape(1, H), g2.reshape(1, H),
                 be2.reshape(1, H), W3, H, H3)

    p3 = _sc_scatter(idx2, y3, H3)
    vuln, typ, conf = _tc_tail(
        p3, y3, dinv2, b3.reshape(1, H3), g3.reshape(1, H3),
        be3.reshape(1, H3), code_embedding, Wf1, bf1.reshape(1, 512),
        Wf2, bf2.reshape(1, 256), Wv, bv.reshape(1, 2),
        Wt, bt.reshape(1, 5), Wc, bc.reshape(1, 1))
    return (vuln, typ, conf)


# speedup vs baseline: 1.1173x; 1.0283x over previous
import jax
import jax.numpy as jnp
from jax import lax
from jax.experimental import pallas as pl
from jax.experimental.pallas import tpu as pltpu
from jax.experimental.pallas import tpu_sc as plsc

N = 10000
NR = 10112
F = 128
H = 128
H3 = 64
CB = 768
E = 320000

BLK = 128
NB = NR // BLK

TBLK = 1264
TNB = NR // TBLK

NCORE = 2
NSUB = 16
TILES = NCORE * NSUB
CHUNK = 120
CH = 84
PT = CH * CHUNK
E2 = TILES * PT
RPS = NR // NSUB
RING = 3
IRING = 6


def _vmesh():
    return plsc.VectorSubcoreMesh(core_axis_name="c", subcore_axis_name="s")


def _sc_deg(idx2):
    def body(idx_hbm, out_hbm, idx_v, ones_v, sv, acc):
        c = lax.axis_index("c")
        s = lax.axis_index("s")
        w = c * NSUB + s
        pltpu.sync_copy(idx_hbm.at[1, w], idx_v)
        for j in range(128 // 16):
            ones_v[pl.ds(j * 16, 16)] = jnp.ones((16,), jnp.float32)
        for j in range(RPS // 16 + 1):
            sv[pl.ds(j * 16, 16)] = jnp.zeros((16,), jnp.float32)
        r0 = s * RPS
        pltpu.sync_copy(sv.at[pl.ds(0, RPS)], acc.at[pl.ds(r0, RPS)])
        plsc.subcore_barrier()

        def step(k, carry):
            pltpu.sync_copy(ones_v.at[pl.ds(0, CHUNK)],
                            acc.at[idx_v.at[k]], add=True)
            return carry

        lax.fori_loop(0, CH, step, 0)
        plsc.subcore_barrier()
        pltpu.sync_copy(acc.at[pl.ds(r0, RPS)], sv.at[pl.ds(0, RPS)])
        pltpu.sync_copy(sv.at[pl.ds(0, RPS)],
                        out_hbm.at[pl.ds(c * NR + r0, RPS)])

    run = pl.kernel(
        body,
        out_type=jax.ShapeDtypeStruct((NCORE * NR,), jnp.float32),
        mesh=_vmesh(),
        scratch_types=[
            pltpu.VMEM((CH, CHUNK), jnp.int32),
            pltpu.VMEM((128,), jnp.float32),
            pltpu.VMEM((RPS + 16,), jnp.float32),
            pltpu.VMEM_SHARED((NR,), jnp.float32),
        ],
    )
    return run(idx2)


def _sc_scatter(idx2, y, h):
    nstage = (NB + NSUB - 1) // NSUB

    def body(idx_hbm, y_hbm, out_hbm, idxv, buf, acc, sem_i, sem_g, sem_s,
             sem_z):
        c = lax.axis_index("c")
        s = lax.axis_index("s")
        w = c * NSUB + s

        def zrow(r, carry):
            for j in range(h // 16):
                buf[r, pl.ds(j * 16, 16)] = jnp.zeros((16,), jnp.float32)
            return carry

        lax.fori_loop(0, BLK, zrow, 0)
        for j in range(nstage):
            blk = s + j * NSUB

            @pl.when(blk < NB)
            def _():
                pltpu.async_copy(buf.at[pl.ds(0, BLK)],
                                 acc.at[pl.ds(blk * BLK, BLK)], sem_z)
        for j in range(nstage):
            blk = s + j * NSUB

            @pl.when(blk < NB)
            def _():
                pltpu.make_async_copy(buf.at[pl.ds(0, BLK)],
                                      acc.at[pl.ds(blk * BLK, BLK)],
                                      sem_z).wait()
        plsc.subcore_barrier()

        def fetch_idx(k):
            bi = lax.rem(k, IRING)
            pltpu.async_copy(idx_hbm.at[:, w, k], idxv.at[bi], sem_i.at[bi])

        def wait_idx(k):
            bi = lax.rem(k, IRING)
            pltpu.make_async_copy(idx_hbm.at[:, w, k], idxv.at[bi],
                                  sem_i.at[bi]).wait()

        def start_gather(k):
            b = lax.rem(k, RING)
            bi = lax.rem(k, IRING)
            pltpu.async_copy(y_hbm.at[idxv.at[bi, 0]],
                             buf.at[pl.ds(b * CHUNK, CHUNK)], sem_g.at[b])

        def wait_gather(k):
            b = lax.rem(k, RING)
            bi = lax.rem(k, IRING)
            pltpu.make_async_copy(y_hbm.at[idxv.at[bi, 0]],
                                  buf.at[pl.ds(b * CHUNK, CHUNK)],
                                  sem_g.at[b]).wait()

        def start_scatter(k):
            b = lax.rem(k, RING)
            bi = lax.rem(k, IRING)
            pltpu.async_copy(buf.at[pl.ds(b * CHUNK, CHUNK)],
                             acc.at[idxv.at[bi, 1]], sem_s.at[b], add=True)

        def wait_scatter(k):
            b = lax.rem(k, RING)
            bi = lax.rem(k, IRING)
            pltpu.make_async_copy(buf.at[pl.ds(b * CHUNK, CHUNK)],
                                  acc.at[idxv.at[bi, 1]], sem_s.at[b]).wait()

        for j in range(4):
            fetch_idx(j)
        for j in range(2):
            wait_idx(j)
            start_gather(j)
        fetch_idx(4)
        wait_gather(0)
        start_scatter(0)
        wait_idx(2)
        start_gather(2)

        def step(k, carry):
            fetch_idx(k + 4)
            wait_gather(k)
            wait_scatter(k - 1)
            wait_idx(k + 2)
            start_gather(k + 2)
            start_scatter(k)
            return carry

        lax.fori_loop(1, CH - 4, step, 0)
        for k in range(CH - 4, CH):
            wait_gather(k)
            start_scatter(k)
            wait_scatter(k - 1)
            if k + 2 < CH:
                wait_idx(k + 2)
                start_gather(k + 2)
        wait_scatter(CH - 1)
        plsc.subcore_barrier()
        for j in range(nstage):
            blk = s + j * NSUB
            half = (j % 2) * BLK

            @pl.when(blk < NB)
            def _():
                if j >= 2:
                    pltpu.make_async_copy(
                        buf.at[pl.ds(half, BLK)],
                        out_hbm.at[c, pl.ds(blk * BLK, BLK)],
                        sem_g.at[j % 2]).wait()
                pltpu.sync_copy(acc.at[pl.ds(blk * BLK, BLK)],
                                buf.at[pl.ds(half, BLK)])
                pltpu.async_copy(buf.at[pl.ds(half, BLK)],
                                 out_hbm.at[c, pl.ds(blk * BLK, BLK)],
                                 sem_g.at[j % 2])
        for j in range(nstage):
            blk = s + j * NSUB
            half = (j % 2) * BLK

            @pl.when((blk < NB) & (blk + 2 * NSUB >= NB))
            def _():
                pltpu.make_async_copy(buf.at[pl.ds(half, BLK)],
                                      out_hbm.at[c, pl.ds(blk * BLK, BLK)],
                                      sem_g.at[j % 2]).wait()

    run = pl.kernel(
        body,
        out_type=jax.ShapeDtypeStruct((NCORE, NR, h), jnp.float32),
        mesh=_vmesh(),
        compiler_params=(pltpu.CompilerParams(use_tc_tiling_on_sc=False)
                         if h % 128 != 0 else None),
        scratch_types=[
            pltpu.VMEM((IRING, 2, CHUNK), jnp.int32),
            pltpu.VMEM((RING * CHUNK, h), jnp.float32),
            pltpu.VMEM_SHARED((NR, h), jnp.float32),
            pltpu.SemaphoreType.DMA((IRING,)),
            pltpu.SemaphoreType.DMA((RING,)),
            pltpu.SemaphoreType.DMA((RING,)),
            pltpu.SemaphoreType.DMA,
        ],
    )
    return run(idx2, y)


def _tc_pre1(xp, d0, d1, W1):
    def body(x_b, d0_b, d1_b, w_b, y_b, dv_b):
        i = pl.program_id(0)
        rows = i * TBLK + lax.broadcasted_iota(jnp.int32, (TBLK, 1), 0)
        d = 1.0 + d0_b[...] + d1_b[...]
        dv = jnp.where(rows < N, lax.rsqrt(d), 0.0)
        dv128 = jnp.broadcast_to(dv, (TBLK, 128))
        dv_b[...] = dv128
        y_b[...] = dv128 * jnp.dot(x_b[...], w_b[...],
                                   preferred_element_type=jnp.float32)

    return pl.pallas_call(
        body,
        grid=(TNB,),
        in_specs=[
            pl.BlockSpec((TBLK, F), lambda i: (i, 0)),
            pl.BlockSpec((TBLK, 1), lambda i: (i, 0)),
            pl.BlockSpec((TBLK, 1), lambda i: (i, 0)),
            pl.BlockSpec((F, H), lambda i: (0, 0)),
        ],
        out_specs=[
            pl.BlockSpec((TBLK, H), lambda i: (i, 0)),
            pl.BlockSpec((TBLK, 128), lambda i: (i, 0)),
        ],
        out_shape=[
            jax.ShapeDtypeStruct((NR, H), jnp.float32),
            jax.ShapeDtypeStruct((NR, 128), jnp.float32),
        ],
    )(xp, d0, d1, W1)


def _tc_mid(pp, y, dinv2, b, g, be, W, hin, hout):
    def body(p_b, y_b, dv_b, b_b, g_b, be_b, w_b, o_b, z_s, dv_s, st_s):
        ph = pl.program_id(0)
        i = pl.program_id(1)

        @pl.when(ph == 0)
        def _():
            dv = dv_b[...]
            dv_s[pl.ds(i * TBLK, TBLK), :] = dv
            z = dv[:, :hin] * (p_b[0] + p_b[1] + y_b[...]) + b_b[...]
            z_s[pl.ds(i * TBLK, TBLK), :] = z
            rows = i * TBLK + lax.broadcasted_iota(jnp.int32, (TBLK, 1), 0)
            zm = jnp.where(rows < N, z, 0.0)
            s1 = jnp.sum(zm, axis=0, keepdims=True)
            s2 = jnp.sum(zm * zm, axis=0, keepdims=True)

            @pl.when(i == 0)
            def _():
                st_s[0:1, :hin] = s1
                st_s[1:2, :hin] = s2

            @pl.when(i != 0)
            def _():
                st_s[0:1, :hin] = st_s[0:1, :hin] + s1
                st_s[1:2, :hin] = st_s[1:2, :hin] + s2

        @pl.when(ph == 1)
        def _():
            mu = st_s[0:1, :hin] * (1.0 / N)
            var = st_s[1:2, :hin] * (1.0 / N) - mu * mu
            kk = g_b[...] * lax.rsqrt(var + 1e-5)
            zb = z_s[pl.ds(i * TBLK, TBLK), :]
            hb = jnp.maximum((zb - mu) * kk + be_b[...], 0.0)
            o_b[...] = dv_s[pl.ds(i * TBLK, TBLK), :hout] * jnp.dot(
                hb, w_b[...], preferred_element_type=jnp.float32)

    return pl.pallas_call(
        body,
        grid=(2, TNB),
        in_specs=[
            pl.BlockSpec((2, TBLK, hin), lambda p, i: (0, i * (1 - p), 0)),
            pl.BlockSpec((TBLK, hin), lambda p, i: (i * (1 - p), 0)),
            pl.BlockSpec((TBLK, 128), lambda p, i: (i * (1 - p), 0)),
            pl.BlockSpec((1, hin), lambda p, i: (0, 0)),
            pl.BlockSpec((1, hin), lambda p, i: (0, 0)),
            pl.BlockSpec((1, hin), lambda p, i: (0, 0)),
            pl.BlockSpec((hin, hout), lambda p, i: (0, 0)),
        ],
        out_specs=pl.BlockSpec((TBLK, hout), lambda p, i: (i * p, 0)),
        out_shape=jax.ShapeDtypeStruct((NR, hout), jnp.float32),
        scratch_shapes=[
            pltpu.VMEM((NR, hin), jnp.float32),
            pltpu.VMEM((NR, 128), jnp.float32),
            pltpu.VMEM((8, 128), jnp.float32),
        ],
    )(pp, y, dinv2, b, g, be, W)


def _tc_tail(pp, y3, dinv2, b3, g3, be3, code, Wf1, bf1, Wf2, bf2,
             Wv, bv, Wt, bt, Wc, bc):
    def body(p_b, y_b, dv_b, b_b, g_b, be_b, code_b, wf1_b, bf1_b,
             wf2_b, bf2_b, wv_b, bv_b, wt_b, bt_b, wc_b, bc_b,
             ov, ot, oc, z_s, st_s):
        i = pl.program_id(0)

        @pl.when(i < TNB)
        def _():
            z = dv_b[...][:, :H3] * (p_b[0] + p_b[1] + y_b[...]) + b_b[...]
            z_s[pl.ds(i * TBLK, TBLK), :] = z
            rows = i * TBLK + lax.broadcasted_iota(jnp.int32, (TBLK, 1), 0)
            zm = jnp.where(rows < N, z, 0.0)
            s1 = jnp.sum(zm, axis=0, keepdims=True)
            s2 = jnp.sum(zm * zm, axis=0, keepdims=True)

            @pl.when(i == 0)
            def _():
                st_s[0:1, :H3] = s1
                st_s[1:2, :H3] = s2

            @pl.when(i != 0)
            def _():
                st_s[0:1, :H3] = st_s[0:1, :H3] + s1
                st_s[1:2, :H3] = st_s[1:2, :H3] + s2

        @pl.when(i == TNB)
        def _():
            mu = st_s[0:1, :H3] * (1.0 / N)
            var = st_s[1:2, :H3] * (1.0 / N) - mu * mu
            kk = g_b[...] * lax.rsqrt(var + 1e-5)
            z = z_s[...]
            rows = lax.broadcasted_iota(jnp.int32, (NR, 1), 0)
            hb = jnp.where(rows < N,
                           jnp.maximum((z - mu) * kk + be_b[...], 0.0), 0.0)
            m = jnp.sum(hb, axis=0, keepdims=True) * (1.0 / N)
            f1 = jnp.maximum(
                jnp.dot(m, wf1_b[0:H3, :], preferred_element_type=jnp.float32)
                + jnp.dot(code_b[...], wf1_b[H3:, :],
                          preferred_element_type=jnp.float32)
                + bf1_b[...], 0.0)
            f2 = jnp.maximum(
                jnp.dot(f1, wf2_b[...], preferred_element_type=jnp.float32)
                + bf2_b[...], 0.0)
            ov[...] = jnp.dot(f2, wv_b[...],
                              preferred_element_type=jnp.float32) + bv_b[...]
            ot[...] = jnp.dot(f2, wt_b[...],
                              preferred_element_type=jnp.float32) + bt_b[...]
            lc = jnp.dot(f2, wc_b[...],
                         preferred_element_type=jnp.float32) + bc_b[...]
            oc[...] = 1.0 / (1.0 + jnp.exp(-lc))

    last = TNB - 1
    return pl.pallas_call(
        body,
        grid=(TNB + 1,),
        in_specs=[
            pl.BlockSpec((2, TBLK, H3), lambda i: (0, jnp.minimum(i, last), 0)),
            pl.BlockSpec((TBLK, H3), lambda i: (jnp.minimum(i, last), 0)),
            pl.BlockSpec((TBLK, 128), lambda i: (jnp.minimum(i, last), 0)),
            pl.BlockSpec((1, H3), lambda i: (0, 0)),
            pl.BlockSpec((1, H3), lambda i: (0, 0)),
            pl.BlockSpec((1, H3), lambda i: (0, 0)),
            pl.BlockSpec((1, CB), lambda i: (0, 0)),
            pl.BlockSpec((H3 + CB, 512), lambda i: (0, 0)),
            pl.BlockSpec((1, 512), lambda i: (0, 0)),
            pl.BlockSpec((512, 256), lambda i: (0, 0)),
            pl.BlockSpec((1, 256), lambda i: (0, 0)),
            pl.BlockSpec((256, 2), lambda i: (0, 0)),
            pl.BlockSpec((1, 2), lambda i: (0, 0)),
            pl.BlockSpec((256, 5), lambda i: (0, 0)),
            pl.BlockSpec((1, 5), lambda i: (0, 0)),
            pl.BlockSpec((256, 1), lambda i: (0, 0)),
            pl.BlockSpec((1, 1), lambda i: (0, 0)),
        ],
        out_specs=[
            pl.BlockSpec((1, 2), lambda i: (0, 0)),
            pl.BlockSpec((1, 5), lambda i: (0, 0)),
            pl.BlockSpec((1, 1), lambda i: (0, 0)),
        ],
        out_shape=[
            jax.ShapeDtypeStruct((1, 2), jnp.float32),
            jax.ShapeDtypeStruct((1, 5), jnp.float32),
            jax.ShapeDtypeStruct((1, 1), jnp.float32),
        ],
        scratch_shapes=[
            pltpu.VMEM((NR, H3), jnp.float32),
            pltpu.VMEM((8, 128), jnp.float32),
        ],
    )(pp, y3, dinv2, b3, g3, be3, code, Wf1, bf1, Wf2, bf2,
      Wv, bv, Wt, bt, Wc, bc)


def kernel(x, edge_index, code_embedding,
           W1, b1, g1, be1, W2, b2, g2, be2, W3, b3, g3, be3,
           Wf1, bf1, Wf2, bf2, Wv, bv, Wt, bt, Wc, bc):
    padn = E2 - E
    ar = jnp.arange(padn, dtype=jnp.int32)
    pad_src = (ar * 13) % N
    pad_dst = N + ar % (NR - N)
    pad2 = jnp.stack([pad_src, pad_dst])
    idx2 = jnp.concatenate([edge_index, pad2], axis=1).reshape(
        2, TILES, CH, CHUNK)

    xp = jnp.pad(x, ((0, NR - N), (0, 0)))

    degp = _sc_deg(idx2).reshape(NCORE, NR)
    d0 = degp[0].reshape(NR, 1)
    d1 = degp[1].reshape(NR, 1)

    y1, dinv2 = _tc_pre1(xp, d0, d1, W1)

    p1 = _sc_scatter(idx2, y1, H)
    y2 = _tc_mid(p1, y1, dinv2, b1.reshape(1, H), g1.reshape(1, H),
                 be1.reshape(1, H), W2, H, H)

    p2 = _sc_scatter(idx2, y2, H)
    y3 = _tc_mid(p2, y2, dinv2, b2.reshape(1, H), g2.reshape(1, H),
                 be2.reshape(1, H), W3, H, H3)

    p3 = _sc_scatter(idx2, y3, H3)
    vuln, typ, conf = _tc_tail(
        p3, y3, dinv2, b3.reshape(1, H3), g3.reshape(1, H3),
        be3.reshape(1, H3), code_embedding, Wf1, bf1.reshape(1, 512),
        Wf2, bf2.reshape(1, 256), Wv, bv.reshape(1, 2),
        Wt, bt.reshape(1, 5), Wc, bc.reshape(1, 1))
    return (vuln, typ, conf)
